# baseline (speedup 1.0000x reference)
import functools

import jax
import jax.numpy as jnp
from jax import lax
from jax.experimental import pallas as pl
from jax.experimental.pallas import tpu as pltpu
from jax.experimental.pallas import tpu_sc as plsc

N0, N1, N2 = 10000, 5000, 2048
E1, E2 = 160000, 65536
D = 128

NC, NS = 2, 16
NT = NC * NS
K1, K2 = 80, 64
ZC = 64
LANES = 16


def _make_sc_segsum(num_rows_pad, chunks_per_tile, K):
    rows_per_tile = num_rows_pad // NS
    assert rows_per_tile % ZC == 0
    cnt_rows = num_rows_pad // 128
    assert cnt_rows <= rows_per_tile
    assert chunks_per_tile % 4 == 0
    hl = 4 if num_rows_pad > 4096 else LANES
    nhalf = LANES // hl

    @functools.partial(
        pl.kernel,
        mesh=plsc.VectorSubcoreMesh(core_axis_name="c", subcore_axis_name="s"),
        compiler_params=pltpu.CompilerParams(needs_layout_passes=False,
                                             use_tc_tiling_on_sc=True),
        out_type=[
            jax.ShapeDtypeStruct((NC, num_rows_pad, D), jnp.float32),
            jax.ShapeDtypeStruct((NT, cnt_rows, 128), jnp.float32),
        ],
        scratch_types=[
            pltpu.VMEM((chunks_per_tile, K), jnp.int32),
            pltpu.VMEM((chunks_per_tile, K), jnp.int32),
            pltpu.VMEM((K, D), jnp.float32),
            pltpu.VMEM((K, D), jnp.float32),
            pltpu.VMEM((K, D), jnp.float32),
            pltpu.VMEM((K, D), jnp.float32),
            pltpu.VMEM((hl * cnt_rows, 128), jnp.float32),
            pltpu.SemaphoreType.DMA,
            pltpu.SemaphoreType.DMA,
            pltpu.SemaphoreType.DMA,
            pltpu.SemaphoreType.DMA,
            pltpu.SemaphoreType.DMA,
            pltpu.SemaphoreType.DMA,
            pltpu.SemaphoreType.DMA,
            pltpu.SemaphoreType.DMA,
            pltpu.SemaphoreType.DMA,
            pltpu.VMEM_SHARED((num_rows_pad, D), jnp.float32),
        ],
    )
    def segsum(table, edges, z_d, acc_out, cnt_out,
               idx_s, idx_d, rows0, rows1, rows2, rows3, hist,
               sg0, sg1, sg2, sg3, ss0, ss1, ss2, ss3, sem_p, acc_sh):
        rows = [rows0, rows1, rows2, rows3]
        sem_g = [sg0, sg1, sg2, sg3]
        sem_s = [ss0, ss1, ss2, ss3]
        c = lax.axis_index("c")
        s = lax.axis_index("s")
        tid = c * NS + s
        pltpu.sync_copy(edges.at[0, pl.ds(tid * chunks_per_tile, chunks_per_tile)], idx_s)
        pltpu.async_copy(edges.at[1, pl.ds(tid * chunks_per_tile, chunks_per_tile)], idx_d, sem_p)

        pltpu.async_copy(table.at[idx_s.at[0]], rows0, sg0)
        pltpu.async_copy(table.at[idx_s.at[1]], rows1, sg1)

        zbase = s * rows_per_tile
        nz = rows_per_tile // ZC
        for i in range(nz):
            pltpu.async_copy(z_d, acc_sh.at[pl.ds(zbase + i * ZC, ZC)], sem_p)
        hist_rows = hl * cnt_rows
        hz = []
        r = 0
        while r < hist_rows:
            n = min(ZC, hist_rows - r)
            hz.append((r, n))
            r += n
        for r, n in hz:
            pltpu.async_copy(z_d.at[pl.ds(0, n)], hist.at[pl.ds(r, n)], sem_p)
        for i in range(nz):
            pltpu.make_async_copy(z_d, acc_sh.at[pl.ds(zbase + i * ZC, ZC)], sem_p).wait()
        for r, n in hz:
            pltpu.make_async_copy(z_d.at[pl.ds(0, n)], hist.at[pl.ds(r, n)], sem_p).wait()
        pltpu.make_async_copy(edges.at[1, pl.ds(tid * chunks_per_tile, chunks_per_tile)],
                              idx_d, sem_p).wait()
        plsc.subcore_barrier()

        iota16 = lax.iota(jnp.int32, LANES)
        lanes = lax.rem(iota16, hl)
        one16 = jnp.ones((LANES,), jnp.float32)
        halves = [None] if nhalf == 1 else [
            (iota16 // hl) == t for t in range(nhalf)]

        lrows = lanes * cnt_rows

        def hist_chunk(j):
            def hbody(g, carry):
                idx16 = idx_d[j, pl.ds(g * LANES, LANES)]
                row = lrows + lax.shift_right_logical(idx16, 7)
                col = lax.bitwise_and(idx16, 127)
                for msk in halves:
                    plsc.addupdate_scatter(hist, [row, col], one16, mask=msk)
                return carry
            lax.fori_loop(0, K // LANES, hbody, 0)

        def waitS(b, j):
            pltpu.make_async_copy(rows[b], acc_sh.at[idx_d.at[j]], sem_s[b]).wait()

        def ebody(jj, carry):
            j0 = jj * 4
            for u in range(4):
                j = j0 + u
                b2 = (u + 2) % 4

                @pl.when(j >= 2)
                def _():
                    waitS(b2, j - 2)

                @pl.when(j + 2 < chunks_per_tile)
                def _():
                    pltpu.async_copy(table.at[idx_s.at[j + 2]], rows[b2], sem_g[b2])

                pltpu.make_async_copy(table.at[idx_s.at[j]], rows[u], sem_g[u]).wait()
                pltpu.async_copy(rows[u], acc_sh.at[idx_d.at[j]], sem_s[u], add=True)
                hist_chunk(j)
            return carry

        lax.fori_loop(0, chunks_per_tile // 4, ebody, 0)
        waitS(2, chunks_per_tile - 2)
        waitS(3, chunks_per_tile - 1)

        def rbody(r, carry):
            for q in range(8):
                o = q * LANES
                tot = hist[r, pl.ds(o, LANES)]
                for l in range(1, hl):
                    tot = tot + hist[l * cnt_rows + r, pl.ds(o, LANES)]
                hist[r, pl.ds(o, LANES)] = tot
            return carry

        lax.fori_loop(0, cnt_rows, rbody, 0)
        plsc.subcore_barrier()

        pltpu.async_copy(acc_sh.at[pl.ds(zbase, rows_per_tile)],
                         acc_out.at[c, pl.ds(zbase, rows_per_tile)], sem_p)
        pltpu.async_copy(hist.at[pl.ds(0, cnt_rows)], cnt_out.at[tid], sem_p)
        pltpu.make_async_copy(acc_sh.at[pl.ds(zbase, rows_per_tile)],
                              acc_out.at[c, pl.ds(zbase, rows_per_tile)], sem_p).wait()
        pltpu.make_async_copy(hist.at[pl.ds(0, cnt_rows)], cnt_out.at[tid], sem_p).wait()

    return segsum


def _tc_layer(acc, cnt, x_src, W_l, b_l, W_r, n_rows, blk, relu, row0=0):
    grid = n_rows // blk
    assert row0 % blk == 0 and blk % 128 == 0
    r0 = row0 // blk
    blkb = blk // 128

    def body(a_ref, c_ref, x_ref, wl_ref, b_ref, wr_ref, o_ref):
        a = a_ref[0] + a_ref[1]
        cb = jnp.sum(c_ref[...], axis=0)
        rowi = lax.broadcasted_iota(jnp.int32, (blk, blkb), 0)
        rsel = (rowi // 128 == lax.broadcasted_iota(jnp.int32, (blk, blkb), 1))
        expand = jnp.dot(rsel.astype(jnp.float32), cb,
                         preferred_element_type=jnp.float32)
        li = lax.broadcasted_iota(jnp.int32, (blk, D), 0) % 128
        lsel = li == lax.broadcasted_iota(jnp.int32, (blk, D), 1)
        crow = jnp.sum(jnp.where(lsel, expand, 0.0), axis=1, keepdims=True)
        mean = a / jnp.maximum(crow, 1.0)
        h = (jnp.dot(mean, wl_ref[...], preferred_element_type=jnp.float32)
             + b_ref[...]
             + jnp.dot(x_ref[...], wr_ref[...], preferred_element_type=jnp.float32))
        if relu:
            h = jnp.maximum(h, 0.0)
        o_ref[...] = h

    return pl.pallas_call(
        body,
        grid=(grid,),
        in_specs=[
            pl.BlockSpec((2, blk, D), lambda i: (0, i + r0, 0)),
            pl.BlockSpec((NT, blkb, 128), lambda i: (0, i + r0, 0)),
            pl.BlockSpec((blk, D), lambda i: (i + r0, 0)),
            pl.BlockSpec((D, D), lambda i: (0, 0)),
            pl.BlockSpec((1, D), lambda i: (0, 0)),
            pl.BlockSpec((D, D), lambda i: (0, 0)),
        ],
        out_specs=pl.BlockSpec((blk, D), lambda i: (i, 0)),
        out_shape=jax.ShapeDtypeStruct((n_rows, D), jnp.float32),
    )(acc, cnt, x_src, W_l, b_l.reshape(1, D), W_r)


EP1 = 163840
P1 = 5120
CPT1 = EP1 // (NT * K1)
P2 = 2048
CPT2 = E2 // (NT * K2)

_segsum1 = _make_sc_segsum(P1, CPT1, K1)
_segsum2 = _make_sc_segsum(P2, CPT2, K2)


def kernel(x, edge_index1, edge_index2, W_l1, b_l1, W_r1, W_l2, b_l2, W_r2):
    z_d = jnp.zeros((ZC, D), jnp.float32)

    e1 = jnp.pad(edge_index1, ((0, 0), (0, EP1 - E1)),
                 constant_values=N1).reshape(2, -1, K1)
    e2 = edge_index2.reshape(2, -1, K2)

    acc1, cnt1 = _segsum1(x, e1, z_d)
    h_head = _tc_layer(acc1, cnt1, x, W_l1, b_l1, W_r1, N2, 1024, relu=True)
    h_tail = _tc_layer(acc1, cnt1, x, W_l1, b_l1, W_r1, P1 - N2, 1024,
                       relu=True, row0=N2)
    acc2, cnt2 = _segsum2(h_head, e2, z_d)
    h = jnp.concatenate([h_head, h_tail])[:N1]
    h2 = _tc_layer(acc2, cnt2, h_head, W_l2, b_l2, W_r2, N2, 1024, relu=False)
    return (h2, h2, h)


# speedup vs baseline: 1.9293x; 1.9293x over previous
import functools

import jax
import jax.numpy as jnp
from jax import lax
from jax.experimental import pallas as pl
from jax.experimental.pallas import tpu as pltpu
from jax.experimental.pallas import tpu_sc as plsc

N0, N1, N2 = 10000, 5000, 2048
E1, E2 = 160000, 65536
D = 128

NC, NS = 2, 16
NT = NC * NS
K1, K2 = 80, 64
ZC = 64
LANES = 16


def _make_sc_segsum(num_rows_pad, chunks_per_tile, K):
    rows_per_tile = num_rows_pad // NS
    assert rows_per_tile % ZC == 0
    cnt_rows = num_rows_pad // 128
    assert cnt_rows <= rows_per_tile
    assert chunks_per_tile % 4 == 0
    hl = 4 if num_rows_pad > 4096 else LANES
    nhalf = LANES // hl

    @functools.partial(
        pl.kernel,
        mesh=plsc.VectorSubcoreMesh(core_axis_name="c", subcore_axis_name="s"),
        compiler_params=pltpu.CompilerParams(needs_layout_passes=False,
                                             use_tc_tiling_on_sc=True),
        out_type=[
            jax.ShapeDtypeStruct((NC, num_rows_pad, D), jnp.float32),
            jax.ShapeDtypeStruct((NT, cnt_rows, 128), jnp.float32),
        ],
        scratch_types=[
            pltpu.VMEM((chunks_per_tile, K), jnp.int32),
            pltpu.VMEM((chunks_per_tile, K), jnp.int32),
            pltpu.VMEM((K, D), jnp.float32),
            pltpu.VMEM((K, D), jnp.float32),
            pltpu.VMEM((K, D), jnp.float32),
            pltpu.VMEM((K, D), jnp.float32),
            pltpu.VMEM((hl * cnt_rows, 128), jnp.float32),
            pltpu.SemaphoreType.DMA,
            pltpu.SemaphoreType.DMA,
            pltpu.SemaphoreType.DMA,
            pltpu.SemaphoreType.DMA,
            pltpu.SemaphoreType.DMA,
            pltpu.SemaphoreType.DMA,
            pltpu.SemaphoreType.DMA,
            pltpu.SemaphoreType.DMA,
            pltpu.SemaphoreType.DMA,
            pltpu.VMEM_SHARED((num_rows_pad, D), jnp.float32),
        ],
    )
    def segsum(table, srcs, dsts, z_d, acc_out, cnt_out,
               idx_s, idx_d, rows0, rows1, rows2, rows3, hist,
               sg0, sg1, sg2, sg3, ss0, ss1, ss2, ss3, sem_p, acc_sh):
        rows = [rows0, rows1, rows2, rows3]
        sem_g = [sg0, sg1, sg2, sg3]
        sem_s = [ss0, ss1, ss2, ss3]
        c = lax.axis_index("c")
        s = lax.axis_index("s")
        tid = c * NS + s
        pltpu.sync_copy(srcs.at[pl.ds(tid * chunks_per_tile, chunks_per_tile)], idx_s)
        pltpu.async_copy(dsts.at[pl.ds(tid * chunks_per_tile, chunks_per_tile)], idx_d, sem_p)

        pltpu.async_copy(table.at[idx_s.at[0]], rows0, sg0)
        pltpu.async_copy(table.at[idx_s.at[1]], rows1, sg1)

        zbase = s * rows_per_tile
        nz = rows_per_tile // ZC
        for i in range(nz):
            pltpu.async_copy(z_d, acc_sh.at[pl.ds(zbase + i * ZC, ZC)], sem_p)
        hist_rows = hl * cnt_rows
        hz = []
        r = 0
        while r < hist_rows:
            n = min(ZC, hist_rows - r)
            hz.append((r, n))
            r += n
        for r, n in hz:
            pltpu.async_copy(z_d.at[pl.ds(0, n)], hist.at[pl.ds(r, n)], sem_p)
        for i in range(nz):
            pltpu.make_async_copy(z_d, acc_sh.at[pl.ds(zbase + i * ZC, ZC)], sem_p).wait()
        for r, n in hz:
            pltpu.make_async_copy(z_d.at[pl.ds(0, n)], hist.at[pl.ds(r, n)], sem_p).wait()
        pltpu.make_async_copy(dsts.at[pl.ds(tid * chunks_per_tile, chunks_per_tile)],
                              idx_d, sem_p).wait()
        plsc.subcore_barrier()

        iota16 = lax.iota(jnp.int32, LANES)
        lanes = lax.rem(iota16, hl)
        one16 = jnp.ones((LANES,), jnp.float32)
        halves = [None] if nhalf == 1 else [
            (iota16 // hl) == t for t in range(nhalf)]

        lrows = lanes * cnt_rows

        def hist_chunk(j):
            def hbody(g, carry):
                idx16 = idx_d[j, pl.ds(g * LANES, LANES)]
                row = lrows + lax.shift_right_logical(idx16, 7)
                col = lax.bitwise_and(idx16, 127)
                for msk in halves:
                    plsc.addupdate_scatter(hist, [row, col], one16, mask=msk)
                return carry
            lax.fori_loop(0, K // LANES, hbody, 0)

        def waitS(b, j):
            pltpu.make_async_copy(rows[b], acc_sh.at[idx_d.at[j]], sem_s[b]).wait()

        def ebody(jj, carry):
            j0 = jj * 4
            for u in range(4):
                j = j0 + u
                b2 = (u + 2) % 4

                @pl.when(j >= 2)
                def _():
                    waitS(b2, j - 2)

                @pl.when(j + 2 < chunks_per_tile)
                def _():
                    pltpu.async_copy(table.at[idx_s.at[j + 2]], rows[b2], sem_g[b2])

                pltpu.make_async_copy(table.at[idx_s.at[j]], rows[u], sem_g[u]).wait()
                pltpu.async_copy(rows[u], acc_sh.at[idx_d.at[j]], sem_s[u], add=True)
                hist_chunk(j)
            return carry

        lax.fori_loop(0, chunks_per_tile // 4, ebody, 0)
        waitS(2, chunks_per_tile - 2)
        waitS(3, chunks_per_tile - 1)

        def rbody(r, carry):
            for q in range(8):
                o = q * LANES
                tot = hist[r, pl.ds(o, LANES)]
                for l in range(1, hl):
                    tot = tot + hist[l * cnt_rows + r, pl.ds(o, LANES)]
                hist[r, pl.ds(o, LANES)] = tot
            return carry

        lax.fori_loop(0, cnt_rows, rbody, 0)
        plsc.subcore_barrier()

        pltpu.async_copy(acc_sh.at[pl.ds(zbase, rows_per_tile)],
                         acc_out.at[c, pl.ds(zbase, rows_per_tile)], sem_p)
        pltpu.async_copy(hist.at[pl.ds(0, cnt_rows)], cnt_out.at[tid], sem_p)
        pltpu.make_async_copy(acc_sh.at[pl.ds(zbase, rows_per_tile)],
                              acc_out.at[c, pl.ds(zbase, rows_per_tile)], sem_p).wait()
        pltpu.make_async_copy(hist.at[pl.ds(0, cnt_rows)], cnt_out.at[tid], sem_p).wait()

    return segsum


def _tc_layer(acc, cnt, x_src, W_l, b_l, W_r, n_rows, blk, relu, row0=0):
    grid = n_rows // blk
    assert row0 % blk == 0 and blk % 128 == 0
    r0 = row0 // blk
    blkb = blk // 128

    def body(a_ref, c_ref, x_ref, wl_ref, b_ref, wr_ref, o_ref):
        a = a_ref[0] + a_ref[1]
        cb = jnp.sum(c_ref[...], axis=0)
        rowi = lax.broadcasted_iota(jnp.int32, (blk, blkb), 0)
        rsel = (rowi // 128 == lax.broadcasted_iota(jnp.int32, (blk, blkb), 1))
        expand = jnp.dot(rsel.astype(jnp.float32), cb,
                         preferred_element_type=jnp.float32)
        li = lax.broadcasted_iota(jnp.int32, (blk, D), 0) % 128
        lsel = li == lax.broadcasted_iota(jnp.int32, (blk, D), 1)
        crow = jnp.sum(jnp.where(lsel, expand, 0.0), axis=1, keepdims=True)
        mean = a / jnp.maximum(crow, 1.0)
        h = (jnp.dot(mean, wl_ref[...], preferred_element_type=jnp.float32)
             + b_ref[...]
             + jnp.dot(x_ref[...], wr_ref[...], preferred_element_type=jnp.float32))
        if relu:
            h = jnp.maximum(h, 0.0)
        o_ref[...] = h

    return pl.pallas_call(
        body,
        grid=(grid,),
        in_specs=[
            pl.BlockSpec((2, blk, D), lambda i: (0, i + r0, 0)),
            pl.BlockSpec((NT, blkb, 128), lambda i: (0, i + r0, 0)),
            pl.BlockSpec((blk, D), lambda i: (i + r0, 0)),
            pl.BlockSpec((D, D), lambda i: (0, 0)),
            pl.BlockSpec((1, D), lambda i: (0, 0)),
            pl.BlockSpec((D, D), lambda i: (0, 0)),
        ],
        out_specs=pl.BlockSpec((blk, D), lambda i: (i, 0)),
        out_shape=jax.ShapeDtypeStruct((n_rows, D), jnp.float32),
    )(acc, cnt, x_src, W_l, b_l.reshape(1, D), W_r)


EP1 = 163840
P1 = 5120
CPT1 = EP1 // (NT * K1)
P2 = 2048
CPT2 = E2 // (NT * K2)

_segsum1 = _make_sc_segsum(P1, CPT1, K1)
_segsum2 = _make_sc_segsum(P2, CPT2, K2)


def kernel(x, edge_index1, edge_index2, W_l1, b_l1, W_r1, W_l2, b_l2, W_r2):
    z_d = jnp.zeros((ZC, D), jnp.float32)

    pad1 = EP1 - E1
    ar = jnp.arange(pad1, dtype=jnp.int32)
    src1 = jnp.concatenate([edge_index1[0], ar % N1]).reshape(-1, K1)
    dst1 = jnp.concatenate([edge_index1[1], N1 + ar % (P1 - N1)]).reshape(-1, K1)
    src2 = edge_index2[0].reshape(-1, K2)
    dst2 = edge_index2[1].reshape(-1, K2)

    acc1, cnt1 = _segsum1(x, src1, dst1, z_d)
    h_head = _tc_layer(acc1, cnt1, x, W_l1, b_l1, W_r1, N2, 1024, relu=True)
    h_tail = _tc_layer(acc1, cnt1, x, W_l1, b_l1, W_r1, P1 - N2, 1024,
                       relu=True, row0=N2)
    acc2, cnt2 = _segsum2(h_head, src2, dst2, z_d)
    h = jnp.concatenate([h_head, h_tail])[:N1]
    h2 = _tc_layer(acc2, cnt2, h_head, W_l2, b_l2, W_r2, N2, 1024, relu=False)
    return (h2, h2, h)
